# Initial kernel scaffold; baseline (speedup 1.0000x reference)
import jax
import jax.numpy as jnp
from jax.experimental import pallas as pl


def kernel(x, edge_index, W0, W1, W2, W3, a0, a1, a2, a3, Wo, ao):
    raise NotImplementedError("write your pallas kernel here")


# speedup vs baseline: 6.1484x; 6.1484x over previous
import jax
import jax.numpy as jnp
from jax import lax
from jax.experimental import pallas as pl
from jax.experimental.pallas import tpu as pltpu
from jax.experimental.pallas import tpu_sc as plsc

N = 10000
D = 128
E = 320000
NC = 2
NS = 16
NW = NC * NS
PN = 10240
PS = 10016
EW = 10240
EPAD = NW * EW
CH = 64
NCHUNK = EW // CH
RPT = PN // NS
L = 16

BR = 1000
GR = N // BR


def _dense1_body(x_ref, wc_ref, ac_ref, h0, h1, h2, h3, s8_ref):
    hb = jnp.dot(x_ref[...], wc_ref[...], preferred_element_type=jnp.float32)
    for i, hr in enumerate((h0, h1, h2, h3)):
        hr[...] = hb[:, i * D:(i + 1) * D]
    s8_ref[...] = jnp.dot(hb, ac_ref[...], preferred_element_type=jnp.float32)


def _dense1(x, wc, ac):
    return pl.pallas_call(
        _dense1_body,
        grid=(GR,),
        in_specs=[
            pl.BlockSpec((BR, D), lambda i: (i, 0)),
            pl.BlockSpec((D, 4 * D), lambda i: (0, 0)),
            pl.BlockSpec((4 * D, 8), lambda i: (0, 0)),
        ],
        out_specs=[pl.BlockSpec((BR, D), lambda i: (i, 0))] * 4
        + [pl.BlockSpec((BR, 8), lambda i: (i, 0))],
        out_shape=[jax.ShapeDtypeStruct((N, D), jnp.float32)] * 4
        + [jax.ShapeDtypeStruct((N, 8), jnp.float32)],
    )(x, wc, ac)


def _dense2_body(n0, n1, n2, n3, d0, d1, d2, d3, wo_ref, ao2_ref,
                 ho_ref, so2_ref):
    cols = []
    for nr, dr in ((n0, d0), (n1, d1), (n2, d2), (n3, d3)):
        nv = nr[...]
        dv = dr[...]
        num = nv[0] + nv[1]
        den = dv[0, :, 0] + dv[1, :, 0]
        hp = num / jnp.maximum(den, 1e-16)[:, None]
        cols.append(jnp.where(hp > 0.0, hp, jnp.exp(hp) - 1.0))
    hc = jnp.concatenate(cols, axis=1)
    ho = jnp.dot(hc, wo_ref[...], preferred_element_type=jnp.float32)
    ho_ref[...] = ho
    so2_ref[...] = jnp.dot(ho, ao2_ref[...], preferred_element_type=jnp.float32)


def _dense2(nums, dens, wo, ao2):
    nspec = pl.BlockSpec((2, BR, D), lambda i: (0, i, 0))
    return pl.pallas_call(
        _dense2_body,
        grid=(GR,),
        in_specs=[nspec] * 8 + [
            pl.BlockSpec((4 * D, D), lambda i: (0, 0)),
            pl.BlockSpec((D, 2), lambda i: (0, 0)),
        ],
        out_specs=[pl.BlockSpec((BR, D), lambda i: (i, 0)),
                   pl.BlockSpec((BR, 2), lambda i: (i, 0))],
        out_shape=[jax.ShapeDtypeStruct((N, D), jnp.float32),
                   jax.ShapeDtypeStruct((N, 2), jnp.float32)],
    )(*nums, *dens, wo, ao2)


def _final_body(no_ref, do_ref, out_ref):
    nv = no_ref[...]
    dv = do_ref[...]
    num = nv[0] + nv[1]
    den = dv[0, :, 0] + dv[1, :, 0]
    hp = num / jnp.maximum(den, 1e-16)[:, None]
    out_ref[...] = jnp.maximum(hp, 0.0)


def _final(num, den):
    return pl.pallas_call(
        _final_body,
        grid=(GR,),
        in_specs=[pl.BlockSpec((2, BR, D), lambda i: (0, i, 0)),
                  pl.BlockSpec((2, BR, D), lambda i: (0, i, 0))],
        out_specs=pl.BlockSpec((BR, D), lambda i: (i, 0)),
        out_shape=jax.ShapeDtypeStruct((N, D), jnp.float32),
    )(num, den)


def _sca_body(s_hbm, earr_hbm, ex_hbm, den_hbm,
              ssrc, sdst, e01c, ex2d, exbuf, den_sh):
    cid = lax.axis_index("c")
    sid = lax.axis_index("s")
    wid = sid * NC + cid
    z16 = jnp.zeros((L,), jnp.float32)
    lane0 = lax.iota(jnp.int32, L) == 0

    def _zr(r, c_):
        for c in range(D // L):
            ex2d[r, pl.ds(c * L, L)] = z16
        return c_
    lax.fori_loop(0, CH, _zr, 0)
    for b in range(RPT // CH):
        pltpu.sync_copy(ex2d, den_sh.at[pl.ds(sid * RPT + b * CH, CH)])

    pltpu.sync_copy(s_hbm.at[0], ssrc)
    pltpu.sync_copy(s_hbm.at[1], sdst)
    plsc.subcore_barrier()

    def _chunk(k, carry):
        pltpu.sync_copy(earr_hbm.at[wid, k], e01c)
        for j in range(CH // L):
            i0 = e01c[0, pl.ds(j * L, L)]
            i1 = e01c[1, pl.ds(j * L, L)]
            s0 = plsc.load_gather(ssrc, [i0])
            s1 = plsc.load_gather(sdst, [i1])
            v = s0 + s1
            ex = jnp.exp(-jnp.where(v > 0.0, v, v * 0.01))
            exbuf[pl.ds(k * CH + j * L, L)] = ex
            for l in range(L):
                ex2d[j * L + l, pl.ds(0, L)] = jnp.where(lane0, ex[l], 0.0)
        pltpu.sync_copy(ex2d, den_sh.at[e01c.at[0]], add=True)
        return carry
    lax.fori_loop(0, NCHUNK, _chunk, 0)

    plsc.subcore_barrier()
    r0 = sid * RPT
    pltpu.sync_copy(den_sh.at[pl.ds(r0, RPT)], den_hbm.at[cid, pl.ds(r0, RPT)])
    pltpu.sync_copy(exbuf, ex_hbm.at[pl.ds(wid * EW, EW)])


_sc_scores = pl.kernel(
    _sca_body,
    out_type=[jax.ShapeDtypeStruct((EPAD,), jnp.float32),
              jax.ShapeDtypeStruct((NC, PN, D), jnp.float32)],
    mesh=plsc.VectorSubcoreMesh(core_axis_name="c", subcore_axis_name="s",
                                num_cores=NC, num_subcores=NS),
    scratch_types=[
        pltpu.VMEM((PS,), jnp.float32),
        pltpu.VMEM((PS,), jnp.float32),
        pltpu.VMEM((2, CH), jnp.int32),
        pltpu.VMEM((CH, D), jnp.float32),
        pltpu.VMEM((EW,), jnp.float32),
        pltpu.VMEM_SHARED((PN, D), jnp.float32),
    ],
    compiler_params=pltpu.CompilerParams(needs_layout_passes=False),
)


def _scb_body(h_hbm, ex_hbm, earr_hbm, num_hbm,
              e01c, exc, rows, num_sh, sem):
    cid = lax.axis_index("c")
    sid = lax.axis_index("s")
    wid = sid * NC + cid
    z16 = jnp.zeros((L,), jnp.float32)

    def _zr(r, c_):
        for c in range(D // L):
            rows[r, pl.ds(c * L, L)] = z16
        return c_
    lax.fori_loop(0, CH, _zr, 0)
    for b in range(RPT // CH):
        pltpu.sync_copy(rows, num_sh.at[pl.ds(sid * RPT + b * CH, CH)])
    plsc.subcore_barrier()

    def _chunk(k, carry):
        pltpu.sync_copy(earr_hbm.at[wid, k], e01c)
        pltpu.sync_copy(ex_hbm.at[pl.ds(wid * EW + k * CH, CH)], exc)
        pltpu.async_copy(h_hbm.at[e01c.at[1]], rows, sem).wait()
        for j in range(CH // L):
            ex = exc[pl.ds(j * L, L)]
            for l in range(L):
                r = j * L + l
                exs = ex[l]
                for c in range(D // L):
                    rows[r, pl.ds(c * L, L)] = rows[r, pl.ds(c * L, L)] * exs
        pltpu.sync_copy(rows, num_sh.at[e01c.at[0]], add=True)
        return carry
    lax.fori_loop(0, NCHUNK, _chunk, 0)

    plsc.subcore_barrier()
    r0 = sid * RPT
    pltpu.sync_copy(num_sh.at[pl.ds(r0, RPT)], num_hbm.at[cid, pl.ds(r0, RPT)])


_sc_agg = pl.kernel(
    _scb_body,
    out_type=jax.ShapeDtypeStruct((NC, PN, D), jnp.float32),
    mesh=plsc.VectorSubcoreMesh(core_axis_name="c", subcore_axis_name="s",
                                num_cores=NC, num_subcores=NS),
    scratch_types=[
        pltpu.VMEM((2, CH), jnp.int32),
        pltpu.VMEM((CH,), jnp.float32),
        pltpu.VMEM((CH, D), jnp.float32),
        pltpu.VMEM_SHARED((PN, D), jnp.float32),
        pltpu.SemaphoreType.DMA,
    ],
    compiler_params=pltpu.CompilerParams(needs_layout_passes=False),
)


def kernel(x, edge_index, W0, W1, W2, W3, a0, a1, a2, a3, Wo, ao):
    f32 = jnp.float32
    wc = jnp.concatenate([W0, W1, W2, W3], axis=1)
    ac = jnp.zeros((4 * D, 8), f32)
    for h, a in enumerate((a0, a1, a2, a3)):
        ac = ac.at[h * D:(h + 1) * D, h].set(a[:D, 0])
        ac = ac.at[h * D:(h + 1) * D, 4 + h].set(a[D:, 0])
    ao2 = jnp.stack([ao[:D, 0], ao[D:, 0]], axis=1)

    pad = EPAD - E
    e0 = jnp.concatenate([edge_index[0],
                          jnp.full((pad,), N, jnp.int32)])
    e1 = jnp.concatenate([edge_index[1],
                          jnp.zeros((pad,), jnp.int32)])
    earr = jnp.stack([e0.reshape(NW, NCHUNK, CH),
                      e1.reshape(NW, NCHUNK, CH)], axis=2)

    h0, h1, h2, h3, s8 = _dense1(x, wc, ac)
    sT = jnp.pad(s8.T, ((0, 0), (0, PS - N)))

    nums, dens = [], []
    for h, hh in enumerate((h0, h1, h2, h3)):
        sp = jnp.stack([sT[h], sT[4 + h]])
        exh, dn = _sc_scores(sp, earr)
        nm = _sc_agg(hh, exh, earr)
        nums.append(nm)
        dens.append(dn)

    ho, so2 = _dense2(nums, dens, Wo, ao2)
    spo = jnp.pad(so2.T, ((0, 0), (0, PS - N)))
    exo, dno = _sc_scores(spo, earr)
    nmo = _sc_agg(ho, exo, earr)
    return _final(nmo, dno)


# speedup vs baseline: 8.9448x; 1.4548x over previous
import jax
import jax.numpy as jnp
from jax import lax
from jax.experimental import pallas as pl
from jax.experimental.pallas import tpu as pltpu
from jax.experimental.pallas import tpu_sc as plsc

N = 10000
D = 128
E = 320000
NC = 2
NS = 16
NW = NC * NS
PN = 10240
PS = 10016
EW = 10240
EPAD = NW * EW
CH = 64
NCHUNK = EW // CH
RPT = PN // NS
L = 16

BR = 1000
GR = N // BR


def _dense1_body(x_ref, wc_ref, ac_ref, h0, h1, h2, h3, s8_ref):
    hb = jnp.dot(x_ref[...], wc_ref[...], preferred_element_type=jnp.float32)
    for i, hr in enumerate((h0, h1, h2, h3)):
        hr[...] = hb[:, i * D:(i + 1) * D]
    s8_ref[...] = jnp.dot(hb, ac_ref[...], preferred_element_type=jnp.float32)


def _dense1(x, wc, ac):
    return pl.pallas_call(
        _dense1_body,
        grid=(GR,),
        in_specs=[
            pl.BlockSpec((BR, D), lambda i: (i, 0)),
            pl.BlockSpec((D, 4 * D), lambda i: (0, 0)),
            pl.BlockSpec((4 * D, 8), lambda i: (0, 0)),
        ],
        out_specs=[pl.BlockSpec((BR, D), lambda i: (i, 0))] * 4
        + [pl.BlockSpec((BR, 8), lambda i: (i, 0))],
        out_shape=[jax.ShapeDtypeStruct((N, D), jnp.float32)] * 4
        + [jax.ShapeDtypeStruct((N, 8), jnp.float32)],
    )(x, wc, ac)


def _dense2_body(n0, n1, n2, n3, d0, d1, d2, d3, wo_ref, ao2_ref,
                 ho_ref, so2_ref):
    cols = []
    for nr, dr in ((n0, d0), (n1, d1), (n2, d2), (n3, d3)):
        nv = nr[...]
        dv = dr[...]
        num = nv[0] + nv[1]
        den = dv[0, :, 0] + dv[1, :, 0]
        hp = num / jnp.maximum(den, 1e-16)[:, None]
        cols.append(jnp.where(hp > 0.0, hp, jnp.exp(hp) - 1.0))
    hc = jnp.concatenate(cols, axis=1)
    ho = jnp.dot(hc, wo_ref[...], preferred_element_type=jnp.float32)
    ho_ref[...] = ho
    so2_ref[...] = jnp.dot(ho, ao2_ref[...], preferred_element_type=jnp.float32)


def _dense2(nums, dens, wo, ao2):
    nspec = pl.BlockSpec((2, BR, D), lambda i: (0, i, 0))
    return pl.pallas_call(
        _dense2_body,
        grid=(GR,),
        in_specs=[nspec] * 8 + [
            pl.BlockSpec((4 * D, D), lambda i: (0, 0)),
            pl.BlockSpec((D, 2), lambda i: (0, 0)),
        ],
        out_specs=[pl.BlockSpec((BR, D), lambda i: (i, 0)),
                   pl.BlockSpec((BR, 2), lambda i: (i, 0))],
        out_shape=[jax.ShapeDtypeStruct((N, D), jnp.float32),
                   jax.ShapeDtypeStruct((N, 2), jnp.float32)],
    )(*nums, *dens, wo, ao2)


def _final_body(no_ref, do_ref, out_ref):
    nv = no_ref[...]
    dv = do_ref[...]
    num = nv[0] + nv[1]
    den = dv[0, :, 0] + dv[1, :, 0]
    hp = num / jnp.maximum(den, 1e-16)[:, None]
    out_ref[...] = jnp.maximum(hp, 0.0)


def _final(num, den):
    return pl.pallas_call(
        _final_body,
        grid=(GR,),
        in_specs=[pl.BlockSpec((2, BR, D), lambda i: (0, i, 0)),
                  pl.BlockSpec((2, BR, D), lambda i: (0, i, 0))],
        out_specs=pl.BlockSpec((BR, D), lambda i: (i, 0)),
        out_shape=jax.ShapeDtypeStruct((N, D), jnp.float32),
    )(num, den)


def _sca_body(s_hbm, earr_hbm, ex_hbm, den_hbm,
              ssrc, sdst, e01c, ex2d, exbuf, den_sh):
    cid = lax.axis_index("c")
    sid = lax.axis_index("s")
    wid = sid * NC + cid
    z16 = jnp.zeros((L,), jnp.float32)
    lane0 = lax.iota(jnp.int32, L) == 0

    def _zr(r, c_):
        for c in range(D // L):
            ex2d[r, pl.ds(c * L, L)] = z16
        return c_
    lax.fori_loop(0, CH, _zr, 0)
    for b in range(RPT // CH):
        pltpu.sync_copy(ex2d, den_sh.at[pl.ds(sid * RPT + b * CH, CH)])

    pltpu.sync_copy(s_hbm.at[0], ssrc)
    pltpu.sync_copy(s_hbm.at[1], sdst)
    plsc.subcore_barrier()

    def _chunk(k, carry):
        pltpu.sync_copy(earr_hbm.at[wid, k], e01c)
        for j in range(CH // L):
            i0 = e01c[0, pl.ds(j * L, L)]
            i1 = e01c[1, pl.ds(j * L, L)]
            s0 = plsc.load_gather(ssrc, [i0])
            s1 = plsc.load_gather(sdst, [i1])
            v = s0 + s1
            ex = jnp.exp(-jnp.where(v > 0.0, v, v * 0.01))
            exbuf[pl.ds(k * CH + j * L, L)] = ex
            for l in range(L):
                ex2d[j * L + l, pl.ds(0, L)] = jnp.where(lane0, ex[l], 0.0)
        pltpu.sync_copy(ex2d, den_sh.at[e01c.at[0]], add=True)
        return carry
    lax.fori_loop(0, NCHUNK, _chunk, 0)

    plsc.subcore_barrier()
    r0 = sid * RPT
    pltpu.sync_copy(den_sh.at[pl.ds(r0, RPT)], den_hbm.at[cid, pl.ds(r0, RPT)])
    pltpu.sync_copy(exbuf, ex_hbm.at[pl.ds(wid * EW, EW)])


_sc_scores = pl.kernel(
    _sca_body,
    out_type=[jax.ShapeDtypeStruct((EPAD,), jnp.float32),
              jax.ShapeDtypeStruct((NC, PN, D), jnp.float32)],
    mesh=plsc.VectorSubcoreMesh(core_axis_name="c", subcore_axis_name="s",
                                num_cores=NC, num_subcores=NS),
    scratch_types=[
        pltpu.VMEM((PS,), jnp.float32),
        pltpu.VMEM((PS,), jnp.float32),
        pltpu.VMEM((2, CH), jnp.int32),
        pltpu.VMEM((CH, D), jnp.float32),
        pltpu.VMEM((EW,), jnp.float32),
        pltpu.VMEM_SHARED((PN, D), jnp.float32),
    ],
    compiler_params=pltpu.CompilerParams(needs_layout_passes=False),
)


CHB = 128
NCHUNKB = EW // CHB


def _scb_body(h_hbm, ex_hbm, earr_hbm, num_hbm,
              e01c0, e01c1, exc0, exc1, rows0, rows1, num_sh, sem0, sem1):
    cid = lax.axis_index("c")
    sid = lax.axis_index("s")
    wid = sid * NC + cid
    z16 = jnp.zeros((L,), jnp.float32)
    bufs = ((e01c0, exc0, rows0, sem0), (e01c1, exc1, rows1, sem1))

    def _zr(r, c_):
        for c in range(D // L):
            rows0[r, pl.ds(c * L, L)] = z16
        return c_
    lax.fori_loop(0, CHB, _zr, 0)
    for b in range(RPT // CHB):
        pltpu.sync_copy(rows0, num_sh.at[pl.ds(sid * RPT + b * CHB, CHB)])
    plsc.subcore_barrier()

    def _fetch(e01c, exc, rows, sem, k):
        pltpu.sync_copy(earr_hbm.at[wid, k], e01c)
        pltpu.sync_copy(ex_hbm.at[pl.ds(wid * EW + k * CHB, CHB)], exc)
        pltpu.async_copy(h_hbm.at[e01c.at[1]], rows, sem)

    for p in (0, 1):
        _fetch(*bufs[p], p)

    def _pair(g, carry):
        for p in (0, 1):
            e01c, exc, rows, sem = bufs[p]
            kk = 2 * g + p
            pltpu.make_async_copy(h_hbm.at[pl.ds(0, CHB)], rows, sem).wait()
            for j in range(CHB // L):
                ex = exc[pl.ds(j * L, L)]
                for l in range(L):
                    r = j * L + l
                    exs = ex[l]
                    for c in range(D // L):
                        rows[r, pl.ds(c * L, L)] = rows[r, pl.ds(c * L, L)] * exs
            pltpu.sync_copy(rows, num_sh.at[e01c.at[0]], add=True)
            _fetch(e01c, exc, rows, sem,
                   jnp.minimum(kk + 2, NCHUNKB - 1))
        return carry
    lax.fori_loop(0, NCHUNKB // 2, _pair, 0)
    for p in (0, 1):
        e01c, exc, rows, sem = bufs[p]
        pltpu.make_async_copy(h_hbm.at[pl.ds(0, CHB)], rows, sem).wait()

    plsc.subcore_barrier()
    r0 = sid * RPT
    pltpu.sync_copy(num_sh.at[pl.ds(r0, RPT)], num_hbm.at[cid, pl.ds(r0, RPT)])


_sc_agg = pl.kernel(
    _scb_body,
    out_type=jax.ShapeDtypeStruct((NC, PN, D), jnp.float32),
    mesh=plsc.VectorSubcoreMesh(core_axis_name="c", subcore_axis_name="s",
                                num_cores=NC, num_subcores=NS),
    scratch_types=[
        pltpu.VMEM((2, CHB), jnp.int32),
        pltpu.VMEM((2, CHB), jnp.int32),
        pltpu.VMEM((CHB,), jnp.float32),
        pltpu.VMEM((CHB,), jnp.float32),
        pltpu.VMEM((CHB, D), jnp.float32),
        pltpu.VMEM((CHB, D), jnp.float32),
        pltpu.VMEM_SHARED((PN, D), jnp.float32),
        pltpu.SemaphoreType.DMA,
        pltpu.SemaphoreType.DMA,
    ],
    compiler_params=pltpu.CompilerParams(needs_layout_passes=False),
)


def kernel(x, edge_index, W0, W1, W2, W3, a0, a1, a2, a3, Wo, ao):
    f32 = jnp.float32
    wc = jnp.concatenate([W0, W1, W2, W3], axis=1)
    ac = jnp.zeros((4 * D, 8), f32)
    for h, a in enumerate((a0, a1, a2, a3)):
        ac = ac.at[h * D:(h + 1) * D, h].set(a[:D, 0])
        ac = ac.at[h * D:(h + 1) * D, 4 + h].set(a[D:, 0])
    ao2 = jnp.stack([ao[:D, 0], ao[D:, 0]], axis=1)

    pad = EPAD - E
    e0 = jnp.concatenate([edge_index[0],
                          jnp.full((pad,), N, jnp.int32)])
    e1 = jnp.concatenate([edge_index[1],
                          jnp.zeros((pad,), jnp.int32)])
    earr = jnp.stack([e0.reshape(NW, NCHUNK, CH),
                      e1.reshape(NW, NCHUNK, CH)], axis=2)
    earrb = jnp.stack([e0.reshape(NW, NCHUNKB, CHB),
                       e1.reshape(NW, NCHUNKB, CHB)], axis=2)

    h0, h1, h2, h3, s8 = _dense1(x, wc, ac)
    sT = jnp.pad(s8.T, ((0, 0), (0, PS - N)))

    nums, dens = [], []
    for h, hh in enumerate((h0, h1, h2, h3)):
        sp = jnp.stack([sT[h], sT[4 + h]])
        exh, dn = _sc_scores(sp, earr)
        nm = _sc_agg(hh, exh, earrb)
        nums.append(nm)
        dens.append(dn)

    ho, so2 = _dense2(nums, dens, Wo, ao2)
    spo = jnp.pad(so2.T, ((0, 0), (0, PS - N)))
    exo, dno = _sc_scores(spo, earr)
    nmo = _sc_agg(ho, exo, earrb)
    return _final(nmo, dno)


# speedup vs baseline: 9.7838x; 1.0938x over previous
import jax
import jax.numpy as jnp
from jax import lax
from jax.experimental import pallas as pl
from jax.experimental.pallas import tpu as pltpu
from jax.experimental.pallas import tpu_sc as plsc

N = 10000
D = 128
E = 320000
NC = 2
NS = 16
NW = NC * NS
PN = 10240
PS = 10016
EW = 10240
EPAD = NW * EW
CH = 64
NCHUNK = EW // CH
RPT = PN // NS
L = 16

BR = 1000
GR = N // BR


def _dense1_body(x_ref, wc_ref, ac_ref, h0, h1, h2, h3, s8_ref):
    hb = jnp.dot(x_ref[...], wc_ref[...], preferred_element_type=jnp.float32)
    for i, hr in enumerate((h0, h1, h2, h3)):
        hr[...] = hb[:, i * D:(i + 1) * D]
    s8_ref[...] = jnp.dot(hb, ac_ref[...], preferred_element_type=jnp.float32)


def _dense1(x, wc, ac):
    return pl.pallas_call(
        _dense1_body,
        grid=(GR,),
        in_specs=[
            pl.BlockSpec((BR, D), lambda i: (i, 0)),
            pl.BlockSpec((D, 4 * D), lambda i: (0, 0)),
            pl.BlockSpec((4 * D, 8), lambda i: (0, 0)),
        ],
        out_specs=[pl.BlockSpec((BR, D), lambda i: (i, 0))] * 4
        + [pl.BlockSpec((BR, 8), lambda i: (i, 0))],
        out_shape=[jax.ShapeDtypeStruct((N, D), jnp.float32)] * 4
        + [jax.ShapeDtypeStruct((N, 8), jnp.float32)],
    )(x, wc, ac)


def _dense2_body(n0, n1, n2, n3, d0, d1, d2, d3, wo_ref, ao2_ref,
                 ho_ref, so2_ref):
    cols = []
    for nr, dr in ((n0, d0), (n1, d1), (n2, d2), (n3, d3)):
        nv = nr[...]
        dv = dr[...]
        num = nv[0] + nv[1]
        den = dv[0, :, 0] + dv[1, :, 0]
        hp = num / jnp.maximum(den, 1e-16)[:, None]
        cols.append(jnp.where(hp > 0.0, hp, jnp.exp(hp) - 1.0))
    hc = jnp.concatenate(cols, axis=1)
    ho = jnp.dot(hc, wo_ref[...], preferred_element_type=jnp.float32)
    ho_ref[...] = ho
    so2_ref[...] = jnp.dot(ho, ao2_ref[...], preferred_element_type=jnp.float32)


def _dense2(nums, dens, wo, ao2):
    nspec = pl.BlockSpec((2, BR, D), lambda i: (0, i, 0))
    return pl.pallas_call(
        _dense2_body,
        grid=(GR,),
        in_specs=[nspec] * 8 + [
            pl.BlockSpec((4 * D, D), lambda i: (0, 0)),
            pl.BlockSpec((D, 2), lambda i: (0, 0)),
        ],
        out_specs=[pl.BlockSpec((BR, D), lambda i: (i, 0)),
                   pl.BlockSpec((BR, 2), lambda i: (i, 0))],
        out_shape=[jax.ShapeDtypeStruct((N, D), jnp.float32),
                   jax.ShapeDtypeStruct((N, 2), jnp.float32)],
    )(*nums, *dens, wo, ao2)


def _final_body(no_ref, do_ref, out_ref):
    nv = no_ref[...]
    dv = do_ref[...]
    num = nv[0] + nv[1]
    den = dv[0, :, 0] + dv[1, :, 0]
    hp = num / jnp.maximum(den, 1e-16)[:, None]
    out_ref[...] = jnp.maximum(hp, 0.0)


def _final(num, den):
    return pl.pallas_call(
        _final_body,
        grid=(GR,),
        in_specs=[pl.BlockSpec((2, BR, D), lambda i: (0, i, 0)),
                  pl.BlockSpec((2, BR, D), lambda i: (0, i, 0))],
        out_specs=pl.BlockSpec((BR, D), lambda i: (i, 0)),
        out_shape=jax.ShapeDtypeStruct((N, D), jnp.float32),
    )(num, den)


def _sca_body(s_hbm, earr_hbm, ex_hbm, den_hbm,
              ssrc, sdst, e01c, exbuf, dloc, dtmp, dred, db, dstage):
    cid = lax.axis_index("c")
    sid = lax.axis_index("s")
    wid = sid * NC + cid
    z16 = jnp.zeros((L,), jnp.float32)
    lane0 = lax.iota(jnp.int32, L) == 0

    def _zd(i, c_):
        dloc[pl.ds(i * L, L)] = z16
        return c_
    lax.fori_loop(0, PN // L, _zd, 0)

    def _zb(r, c_):
        for c in range(D // L):
            db[r, pl.ds(c * L, L)] = z16
        return c_
    lax.fori_loop(0, 64, _zb, 0)

    pltpu.sync_copy(s_hbm.at[0], ssrc)
    pltpu.sync_copy(s_hbm.at[1], sdst)

    def _chunk(k, carry):
        pltpu.sync_copy(earr_hbm.at[wid, k], e01c)
        for j in range(CH // L):
            i0 = e01c[0, pl.ds(j * L, L)]
            i1 = e01c[1, pl.ds(j * L, L)]
            s0 = plsc.load_gather(ssrc, [i0])
            s1 = plsc.load_gather(sdst, [i1])
            v = s0 + s1
            ex = jnp.exp(-jnp.where(v > 0.0, v, v * 0.01))
            exbuf[pl.ds(k * CH + j * L, L)] = ex
            plsc.addupdate_scatter(dloc, [i0], ex)
        return carry
    lax.fori_loop(0, NCHUNK, _chunk, 0)

    pltpu.sync_copy(dloc, dstage.at[sid])
    plsc.subcore_barrier()
    c0 = sid * RPT

    def _zr2(i, c_):
        dred[pl.ds(i * L, L)] = z16
        return c_
    lax.fori_loop(0, RPT // L, _zr2, 0)
    for t in range(NS):
        pltpu.sync_copy(dstage.at[t, pl.ds(c0, RPT)], dtmp)

        def _acc(i, c_):
            dred[pl.ds(i * L, L)] = (dred[pl.ds(i * L, L)]
                                     + dtmp[pl.ds(i * L, L)])
            return c_
        lax.fori_loop(0, RPT // L, _acc, 0)
    for b in range(RPT // 64):
        for g in range(4):
            dv = dred[pl.ds(b * 64 + g * L, L)]
            for l in range(L):
                db[g * L + l, pl.ds(0, L)] = jnp.where(lane0, dv[l], 0.0)
        pltpu.sync_copy(db, den_hbm.at[cid, pl.ds(sid * RPT + b * 64, 64)])
    pltpu.sync_copy(exbuf, ex_hbm.at[pl.ds(wid * EW, EW)])


_sc_scores = pl.kernel(
    _sca_body,
    out_type=[jax.ShapeDtypeStruct((EPAD,), jnp.float32),
              jax.ShapeDtypeStruct((NC, PN, D), jnp.float32)],
    mesh=plsc.VectorSubcoreMesh(core_axis_name="c", subcore_axis_name="s",
                                num_cores=NC, num_subcores=NS),
    scratch_types=[
        pltpu.VMEM((PS,), jnp.float32),
        pltpu.VMEM((PS,), jnp.float32),
        pltpu.VMEM((2, CH), jnp.int32),
        pltpu.VMEM((EW,), jnp.float32),
        pltpu.VMEM((PN,), jnp.float32),
        pltpu.VMEM((RPT,), jnp.float32),
        pltpu.VMEM((RPT,), jnp.float32),
        pltpu.VMEM((64, D), jnp.float32),
        pltpu.VMEM_SHARED((NS, PN), jnp.float32),
    ],
    compiler_params=pltpu.CompilerParams(needs_layout_passes=False),
)


CHB = 128
NCHUNKB = EW // CHB


def _scb_body(h_hbm, ex_hbm, earr_hbm, num_hbm,
              e01c0, e01c1, exc0, exc1, rows0, rows1, num_sh, sem0, sem1):
    cid = lax.axis_index("c")
    sid = lax.axis_index("s")
    wid = sid * NC + cid
    z16 = jnp.zeros((L,), jnp.float32)
    bufs = ((e01c0, exc0, rows0, sem0), (e01c1, exc1, rows1, sem1))

    def _zr(r, c_):
        for c in range(D // L):
            rows0[r, pl.ds(c * L, L)] = z16
        return c_
    lax.fori_loop(0, CHB, _zr, 0)
    for b in range(RPT // CHB):
        pltpu.sync_copy(rows0, num_sh.at[pl.ds(sid * RPT + b * CHB, CHB)])
    plsc.subcore_barrier()

    def _fetch(e01c, exc, rows, sem, k):
        pltpu.sync_copy(earr_hbm.at[wid, k], e01c)
        pltpu.sync_copy(ex_hbm.at[pl.ds(wid * EW + k * CHB, CHB)], exc)
        pltpu.async_copy(h_hbm.at[e01c.at[1]], rows, sem)

    for p in (0, 1):
        _fetch(*bufs[p], p)

    def _pair(g, carry):
        for p in (0, 1):
            e01c, exc, rows, sem = bufs[p]
            kk = 2 * g + p
            pltpu.make_async_copy(h_hbm.at[pl.ds(0, CHB)], rows, sem).wait()
            for j in range(CHB // L):
                ex = exc[pl.ds(j * L, L)]
                for l in range(L):
                    r = j * L + l
                    exs = ex[l]
                    for c in range(D // L):
                        rows[r, pl.ds(c * L, L)] = rows[r, pl.ds(c * L, L)] * exs
            pltpu.sync_copy(rows, num_sh.at[e01c.at[0]], add=True)
            _fetch(e01c, exc, rows, sem,
                   jnp.minimum(kk + 2, NCHUNKB - 1))
        return carry
    lax.fori_loop(0, NCHUNKB // 2, _pair, 0)
    for p in (0, 1):
        e01c, exc, rows, sem = bufs[p]
        pltpu.make_async_copy(h_hbm.at[pl.ds(0, CHB)], rows, sem).wait()

    plsc.subcore_barrier()
    r0 = sid * RPT
    pltpu.sync_copy(num_sh.at[pl.ds(r0, RPT)], num_hbm.at[cid, pl.ds(r0, RPT)])


_sc_agg = pl.kernel(
    _scb_body,
    out_type=jax.ShapeDtypeStruct((NC, PN, D), jnp.float32),
    mesh=plsc.VectorSubcoreMesh(core_axis_name="c", subcore_axis_name="s",
                                num_cores=NC, num_subcores=NS),
    scratch_types=[
        pltpu.VMEM((2, CHB), jnp.int32),
        pltpu.VMEM((2, CHB), jnp.int32),
        pltpu.VMEM((CHB,), jnp.float32),
        pltpu.VMEM((CHB,), jnp.float32),
        pltpu.VMEM((CHB, D), jnp.float32),
        pltpu.VMEM((CHB, D), jnp.float32),
        pltpu.VMEM_SHARED((PN, D), jnp.float32),
        pltpu.SemaphoreType.DMA,
        pltpu.SemaphoreType.DMA,
    ],
    compiler_params=pltpu.CompilerParams(needs_layout_passes=False),
)


def kernel(x, edge_index, W0, W1, W2, W3, a0, a1, a2, a3, Wo, ao):
    f32 = jnp.float32
    wc = jnp.concatenate([W0, W1, W2, W3], axis=1)
    ac = jnp.zeros((4 * D, 8), f32)
    for h, a in enumerate((a0, a1, a2, a3)):
        ac = ac.at[h * D:(h + 1) * D, h].set(a[:D, 0])
        ac = ac.at[h * D:(h + 1) * D, 4 + h].set(a[D:, 0])
    ao2 = jnp.stack([ao[:D, 0], ao[D:, 0]], axis=1)

    pad = EPAD - E
    e0 = jnp.concatenate([edge_index[0],
                          jnp.full((pad,), N, jnp.int32)])
    e1 = jnp.concatenate([edge_index[1],
                          jnp.zeros((pad,), jnp.int32)])
    earr = jnp.stack([e0.reshape(NW, NCHUNK, CH),
                      e1.reshape(NW, NCHUNK, CH)], axis=2)
    earrb = jnp.stack([e0.reshape(NW, NCHUNKB, CHB),
                       e1.reshape(NW, NCHUNKB, CHB)], axis=2)

    h0, h1, h2, h3, s8 = _dense1(x, wc, ac)
    sT = jnp.pad(s8.T, ((0, 0), (0, PS - N)))

    nums, dens = [], []
    for h, hh in enumerate((h0, h1, h2, h3)):
        sp = jnp.stack([sT[h], sT[4 + h]])
        exh, dn = _sc_scores(sp, earr)
        nm = _sc_agg(hh, exh, earrb)
        nums.append(nm)
        dens.append(dn)

    ho, so2 = _dense2(nums, dens, Wo, ao2)
    spo = jnp.pad(so2.T, ((0, 0), (0, PS - N)))
    exo, dno = _sc_scores(spo, earr)
    nmo = _sc_agg(ho, exo, earrb)
    return _final(nmo, dno)
